# baseline (speedup 1.0000x reference)
import functools

import jax
import jax.numpy as jnp
from jax import lax
from jax.experimental import pallas as pl
from jax.experimental.pallas import tpu as pltpu
from jax.experimental.pallas import tpu_sc as plsc

NEG_INF = float("-inf")
BIG_I32 = 2**31 - 1
EPS = 1e-8
TOPK = 10
NPOP = 11
SEG = 128
KC = 8192
SEG_PER_KC = KC // SEG
NSLOT = 16


def _phase_a_kernel(q_ref, m_ref, sims_ref, seg_ref, rv_ref, ri_ref,
                    *, k_real, nkc, qb):
    j = pl.program_id(0)
    i = pl.program_id(1)
    rv_ref = rv_ref.at[pl.ds(i * qb, qb), :]
    ri_ref = ri_ref.at[pl.ds(i * qb, qb), :]

    @pl.when(j == 0)
    def _init():
        rv_ref[...] = jnp.full((qb, NSLOT), NEG_INF, jnp.float32)
        ri_ref[...] = jnp.zeros((qb, NSLOT), jnp.int32)

    qn = q_ref[...]
    mn = m_ref[...]
    sims = jax.lax.dot_general(
        qn, mn, (((1,), (1,)), ((), ())),
        preferred_element_type=jnp.float32)
    c = j

    maxes = []
    for s in range(SEG_PER_KC):
        piece = sims[:, s * SEG:(s + 1) * SEG]
        sims_ref[:, s * 8:(s + 1) * 8, :] = piece.reshape(qb // 8, 8, SEG)
        maxes.append(jnp.max(piece, axis=1, keepdims=True))
    segm = jnp.concatenate(maxes, axis=1)

    gseg = c * SEG_PER_KC + jax.lax.broadcasted_iota(
        jnp.int32, (qb, SEG_PER_KC), 1)
    segm = jnp.where(gseg * SEG < k_real, segm, NEG_INF)
    aug_v = jnp.concatenate([rv_ref[...], segm], axis=1)
    aug_i = jnp.concatenate([ri_ref[...], gseg], axis=1)
    lane = jax.lax.broadcasted_iota(jnp.int32, (qb, NSLOT), 1)

    new_v = jnp.full((qb, NSLOT), NEG_INF, jnp.float32)
    new_i = jnp.zeros((qb, NSLOT), jnp.int32)
    for i in range(NPOP):
        mx = jnp.max(aug_v, axis=1, keepdims=True)
        sel = jnp.min(jnp.where(aug_v == mx, aug_i, BIG_I32), axis=1,
                      keepdims=True)
        aug_v = jnp.where(aug_i == sel, NEG_INF, aug_v)
        new_v = jnp.where(lane == i, mx, new_v)
        new_i = jnp.where(lane == i, sel, new_i)

    rv_ref[...] = new_v
    ri_ref[...] = new_i

    @pl.when(j == nkc - 1)
    def _emit():
        seg_ref[...] = new_i


def _gather_sc(table, seg_ids, nseg):
    q_total = seg_ids.shape[0]
    info = plsc.get_sparse_core_info()
    nc, ns = info.num_cores, info.num_subcores
    nw = nc * ns
    qpw = q_total // nw
    row_stride = nseg * 8
    mesh = plsc.VectorSubcoreMesh(core_axis_name="c", subcore_axis_name="s")

    @functools.partial(
        pl.kernel, mesh=mesh,
        out_type=jax.ShapeDtypeStruct((q_total * NSLOT, SEG), jnp.float32),
        scratch_types=[
            pltpu.VMEM((qpw, NSLOT), jnp.int32),
            pltpu.VMEM((qpw * NSLOT, SEG), jnp.float32),
            pltpu.SemaphoreType.DMA,
        ],
    )
    def k(table_hbm, seg_hbm, out_hbm, segv, rows, sem):
        wid = lax.axis_index("s") * nc + lax.axis_index("c")
        qbase = wid * qpw
        pltpu.sync_copy(seg_hbm.at[pl.ds(qbase, qpw)], segv)
        copies = []
        for i in range(qpw):
            q = qbase + i
            flat = segv[i, :] * 8 + ((q // 8) * row_stride + q % 8)
            copies.append(pltpu.async_copy(
                table_hbm.at[flat],
                rows.at[pl.ds(i * NSLOT, NSLOT), :], sem))
        for c in copies:
            c.wait()
        pltpu.sync_copy(rows, out_hbm.at[pl.ds(qbase * NSLOT, qpw * NSLOT)])

    return k(table, seg_ids).reshape(q_total, NSLOT, SEG)


def _phase_d_kernel(g_ref, seg_ref, vals_ref, idx_ref, *, k_real):
    qb = seg_ref.shape[0]
    seg = seg_ref[...]
    lane = jax.lax.broadcasted_iota(jnp.int32, (qb, SEG), 1)
    g3 = g_ref[...]
    cand = jnp.concatenate([g3[:, t, :] for t in range(NPOP)], axis=1)
    gidx = jnp.concatenate(
        [seg[:, t:t + 1] * SEG + lane for t in range(NPOP)], axis=1)
    cand = jnp.where(gidx < k_real, cand, NEG_INF)
    vals = []
    idxs = []
    for i in range(TOPK):
        mx = jnp.max(cand, axis=1, keepdims=True)
        sel = jnp.min(jnp.where(cand == mx, gidx, BIG_I32), axis=1,
                      keepdims=True)
        cand = jnp.where(gidx == sel, NEG_INF, cand)
        vals.append(mx)
        idxs.append(sel)
    vals_ref[...] = jnp.concatenate(vals, axis=1)
    idx_ref[...] = jnp.concatenate(idxs, axis=1)


def kernel(query, memory_embeddings, top_k):
    del top_k
    q_total, d = query.shape
    k_real = memory_embeddings.shape[0]
    nkc = -(-k_real // KC)
    k_pad = nkc * KC
    nseg = k_pad // SEG
    if k_pad != k_real:
        mem = jnp.pad(memory_embeddings, ((0, k_pad - k_real), (0, 0)))
    else:
        mem = memory_embeddings
    query = query / jnp.maximum(
        jnp.linalg.norm(query, axis=-1, keepdims=True), EPS)
    mem = mem / jnp.maximum(
        jnp.linalg.norm(mem, axis=-1, keepdims=True), EPS)
    qb = min(256, q_total)

    sims3, seg_ids = pl.pallas_call(
        functools.partial(_phase_a_kernel, k_real=k_real, nkc=nkc, qb=qb),
        grid=(nkc, q_total // qb),
        in_specs=[
            pl.BlockSpec((qb, d), lambda j, i: (i, 0)),
            pl.BlockSpec((KC, d), lambda j, i: (j, 0)),
        ],
        out_specs=[
            pl.BlockSpec((qb // 8, SEG_PER_KC * 8, SEG),
                         lambda j, i: (i, j, 0)),
            pl.BlockSpec((qb, NSLOT), lambda j, i: (i, 0)),
        ],
        out_shape=[
            jax.ShapeDtypeStruct((q_total // 8, nseg * 8, SEG), jnp.float32),
            jax.ShapeDtypeStruct((q_total, NSLOT), jnp.int32),
        ],
        scratch_shapes=[
            pltpu.VMEM((q_total, NSLOT), jnp.float32),
            pltpu.VMEM((q_total, NSLOT), jnp.int32),
        ],
    )(query, mem)

    table = sims3.reshape(q_total * nseg, SEG)
    g = _gather_sc(table, seg_ids, nseg)

    vals, idx = pl.pallas_call(
        functools.partial(_phase_d_kernel, k_real=k_real),
        grid=(q_total // qb,),
        in_specs=[
            pl.BlockSpec((qb, NSLOT, SEG), lambda i: (i, 0, 0)),
            pl.BlockSpec((qb, NSLOT), lambda i: (i, 0)),
        ],
        out_specs=[
            pl.BlockSpec((qb, TOPK), lambda i: (i, 0)),
            pl.BlockSpec((qb, TOPK), lambda i: (i, 0)),
        ],
        out_shape=[
            jax.ShapeDtypeStruct((q_total, TOPK), jnp.float32),
            jax.ShapeDtypeStruct((q_total, TOPK), jnp.int32),
        ],
    )(g, seg_ids)
    return vals, idx


# speedup vs baseline: 1.0997x; 1.0997x over previous
import functools

import jax
import jax.numpy as jnp
from jax import lax
from jax.experimental import pallas as pl
from jax.experimental.pallas import tpu as pltpu
from jax.experimental.pallas import tpu_sc as plsc

NEG_INF = float("-inf")
BIG_I32 = 2**31 - 1
EPS = 1e-8
TOPK = 10
NPOP = 11
SEG = 128
KC = 12288
SEG_PER_KC = KC // SEG
NSLOT = 16


def _phase_a_kernel(q_ref, m_ref, sims_ref, seg_ref, rv_ref, ri_ref,
                    *, k_real, nkc, qb):
    j = pl.program_id(0)
    i = pl.program_id(1)
    rv_ref = rv_ref.at[pl.ds(i * qb, qb), :]
    ri_ref = ri_ref.at[pl.ds(i * qb, qb), :]

    @pl.when(j == 0)
    def _init():
        rv_ref[...] = jnp.full((qb, NSLOT), NEG_INF, jnp.float32)
        ri_ref[...] = jnp.zeros((qb, NSLOT), jnp.int32)

    qn = q_ref[...]
    mn = m_ref[...]
    sims = jax.lax.dot_general(
        qn, mn, (((1,), (1,)), ((), ())),
        preferred_element_type=jnp.float32)
    c = j

    maxes = []
    for s in range(SEG_PER_KC):
        piece = sims[:, s * SEG:(s + 1) * SEG]
        sims_ref[:, s * 8:(s + 1) * 8, :] = piece.reshape(qb // 8, 8, SEG)
        maxes.append(jnp.max(piece, axis=1, keepdims=True))
    segm = jnp.concatenate(maxes, axis=1)

    gseg = c * SEG_PER_KC + jax.lax.broadcasted_iota(
        jnp.int32, (qb, SEG_PER_KC), 1)
    segm = jnp.where(gseg * SEG < k_real, segm, NEG_INF)
    aug_v = jnp.concatenate([rv_ref[...], segm], axis=1)
    aug_i = jnp.concatenate([ri_ref[...], gseg], axis=1)
    lane = jax.lax.broadcasted_iota(jnp.int32, (qb, NSLOT), 1)

    new_v = jnp.full((qb, NSLOT), NEG_INF, jnp.float32)
    new_i = jnp.zeros((qb, NSLOT), jnp.int32)
    for i in range(NPOP):
        mx = jnp.max(aug_v, axis=1, keepdims=True)
        sel = jnp.min(jnp.where(aug_v == mx, aug_i, BIG_I32), axis=1,
                      keepdims=True)
        aug_v = jnp.where(aug_i == sel, NEG_INF, aug_v)
        new_v = jnp.where(lane == i, mx, new_v)
        new_i = jnp.where(lane == i, sel, new_i)

    rv_ref[...] = new_v
    ri_ref[...] = new_i

    @pl.when(j == nkc - 1)
    def _emit():
        seg_ref[...] = new_i


def _gather_sc(table, seg_ids, nseg):
    q_total = seg_ids.shape[0]
    info = plsc.get_sparse_core_info()
    nc, ns = info.num_cores, info.num_subcores
    nw = nc * ns
    qpw = q_total // nw
    row_stride = nseg * 8
    mesh = plsc.VectorSubcoreMesh(core_axis_name="c", subcore_axis_name="s")

    @functools.partial(
        pl.kernel, mesh=mesh,
        out_type=jax.ShapeDtypeStruct((q_total * NSLOT, SEG), jnp.float32),
        scratch_types=[
            pltpu.VMEM((qpw, NSLOT), jnp.int32),
            pltpu.VMEM((qpw * NSLOT, SEG), jnp.float32),
            pltpu.SemaphoreType.DMA,
        ],
    )
    def k(table_hbm, seg_hbm, out_hbm, segv, rows, sem):
        wid = lax.axis_index("s") * nc + lax.axis_index("c")
        qbase = wid * qpw
        pltpu.sync_copy(seg_hbm.at[pl.ds(qbase, qpw)], segv)
        copies = []
        for i in range(qpw):
            q = qbase + i
            flat = segv[i, :] * 8 + ((q // 8) * row_stride + q % 8)
            copies.append(pltpu.async_copy(
                table_hbm.at[flat],
                rows.at[pl.ds(i * NSLOT, NSLOT), :], sem))
        for c in copies:
            c.wait()
        pltpu.sync_copy(rows, out_hbm.at[pl.ds(qbase * NSLOT, qpw * NSLOT)])

    return k(table, seg_ids).reshape(q_total, NSLOT, SEG)


def _phase_d_kernel(g_ref, seg_ref, vals_ref, idx_ref, *, k_real):
    qb = seg_ref.shape[0]
    seg = seg_ref[...]
    lane = jax.lax.broadcasted_iota(jnp.int32, (qb, SEG), 1)
    g3 = g_ref[...]
    cand = jnp.concatenate([g3[:, t, :] for t in range(NPOP)], axis=1)
    gidx = jnp.concatenate(
        [seg[:, t:t + 1] * SEG + lane for t in range(NPOP)], axis=1)
    cand = jnp.where(gidx < k_real, cand, NEG_INF)
    vals = []
    idxs = []
    for i in range(TOPK):
        mx = jnp.max(cand, axis=1, keepdims=True)
        sel = jnp.min(jnp.where(cand == mx, gidx, BIG_I32), axis=1,
                      keepdims=True)
        cand = jnp.where(gidx == sel, NEG_INF, cand)
        vals.append(mx)
        idxs.append(sel)
    vals_ref[...] = jnp.concatenate(vals, axis=1)
    idx_ref[...] = jnp.concatenate(idxs, axis=1)


def kernel(query, memory_embeddings, top_k):
    del top_k
    q_total, d = query.shape
    k_real = memory_embeddings.shape[0]
    nkc = -(-k_real // KC)
    k_pad = nkc * KC
    nseg = k_pad // SEG
    if k_pad != k_real:
        mem = jnp.pad(memory_embeddings, ((0, k_pad - k_real), (0, 0)))
    else:
        mem = memory_embeddings
    query = query / jnp.maximum(
        jnp.linalg.norm(query, axis=-1, keepdims=True), EPS)
    mem = mem / jnp.maximum(
        jnp.linalg.norm(mem, axis=-1, keepdims=True), EPS)
    qb = min(256, q_total)

    sims3, seg_ids = pl.pallas_call(
        functools.partial(_phase_a_kernel, k_real=k_real, nkc=nkc, qb=qb),
        grid=(nkc, q_total // qb),
        in_specs=[
            pl.BlockSpec((qb, d), lambda j, i: (i, 0)),
            pl.BlockSpec((KC, d), lambda j, i: (j, 0)),
        ],
        out_specs=[
            pl.BlockSpec((qb // 8, SEG_PER_KC * 8, SEG),
                         lambda j, i: (i, j, 0)),
            pl.BlockSpec((qb, NSLOT), lambda j, i: (i, 0)),
        ],
        out_shape=[
            jax.ShapeDtypeStruct((q_total // 8, nseg * 8, SEG), jnp.float32),
            jax.ShapeDtypeStruct((q_total, NSLOT), jnp.int32),
        ],
        scratch_shapes=[
            pltpu.VMEM((q_total, NSLOT), jnp.float32),
            pltpu.VMEM((q_total, NSLOT), jnp.int32),
        ],
    )(query, mem)

    table = sims3.reshape(q_total * nseg, SEG)
    g = _gather_sc(table, seg_ids, nseg)

    vals, idx = pl.pallas_call(
        functools.partial(_phase_d_kernel, k_real=k_real),
        grid=(q_total // qb,),
        in_specs=[
            pl.BlockSpec((qb, NSLOT, SEG), lambda i: (i, 0, 0)),
            pl.BlockSpec((qb, NSLOT), lambda i: (i, 0)),
        ],
        out_specs=[
            pl.BlockSpec((qb, TOPK), lambda i: (i, 0)),
            pl.BlockSpec((qb, TOPK), lambda i: (i, 0)),
        ],
        out_shape=[
            jax.ShapeDtypeStruct((q_total, TOPK), jnp.float32),
            jax.ShapeDtypeStruct((q_total, TOPK), jnp.int32),
        ],
    )(g, seg_ids)
    return vals, idx


# speedup vs baseline: 1.1363x; 1.0333x over previous
import functools

import jax
import jax.numpy as jnp
from jax import lax
from jax.experimental import pallas as pl
from jax.experimental.pallas import tpu as pltpu
from jax.experimental.pallas import tpu_sc as plsc

NEG_INF = float("-inf")
BIG_I32 = 2**31 - 1
EPS = 1e-8
TOPK = 10
NPOP = 11
SEG = 128
KC = 10240
SEG_PER_KC = KC // SEG
NSLOT = 16


def _phase_a_kernel(q_ref, m_ref, sims_ref, seg_ref, rv_ref, ri_ref,
                    *, k_real, nkc, qb):
    j = pl.program_id(0)
    i = pl.program_id(1)
    rv_ref = rv_ref.at[pl.ds(i * qb, qb), :]
    ri_ref = ri_ref.at[pl.ds(i * qb, qb), :]

    @pl.when(j == 0)
    def _init():
        rv_ref[...] = jnp.full((qb, NSLOT), NEG_INF, jnp.float32)
        ri_ref[...] = jnp.zeros((qb, NSLOT), jnp.int32)

    qn = q_ref[...]
    mn = m_ref[...]
    sims = jax.lax.dot_general(
        qn, mn, (((1,), (1,)), ((), ())),
        preferred_element_type=jnp.float32)
    c = j

    maxes = []
    for s in range(SEG_PER_KC):
        piece = sims[:, s * SEG:(s + 1) * SEG]
        sims_ref[:, s * 8:(s + 1) * 8, :] = piece.reshape(qb // 8, 8, SEG)
        maxes.append(jnp.max(piece, axis=1, keepdims=True))
    segm = jnp.concatenate(maxes, axis=1)

    gseg = c * SEG_PER_KC + jax.lax.broadcasted_iota(
        jnp.int32, (qb, SEG_PER_KC), 1)
    segm = jnp.where(gseg * SEG < k_real, segm, NEG_INF)
    aug_v = jnp.concatenate([rv_ref[...], segm], axis=1)
    aug_i = jnp.concatenate([ri_ref[...], gseg], axis=1)
    lane = jax.lax.broadcasted_iota(jnp.int32, (qb, NSLOT), 1)

    new_v = jnp.full((qb, NSLOT), NEG_INF, jnp.float32)
    new_i = jnp.zeros((qb, NSLOT), jnp.int32)
    for i in range(NPOP):
        mx = jnp.max(aug_v, axis=1, keepdims=True)
        sel = jnp.min(jnp.where(aug_v == mx, aug_i, BIG_I32), axis=1,
                      keepdims=True)
        aug_v = jnp.where(aug_i == sel, NEG_INF, aug_v)
        new_v = jnp.where(lane == i, mx, new_v)
        new_i = jnp.where(lane == i, sel, new_i)

    rv_ref[...] = new_v
    ri_ref[...] = new_i

    @pl.when(j == nkc - 1)
    def _emit():
        seg_ref[...] = new_i


def _gather_sc(table, seg_ids, nseg):
    q_total = seg_ids.shape[0]
    info = plsc.get_sparse_core_info()
    nc, ns = info.num_cores, info.num_subcores
    nw = nc * ns
    qpw = q_total // nw
    row_stride = nseg * 8
    mesh = plsc.VectorSubcoreMesh(core_axis_name="c", subcore_axis_name="s")

    @functools.partial(
        pl.kernel, mesh=mesh,
        out_type=jax.ShapeDtypeStruct((q_total * NSLOT, SEG), jnp.float32),
        scratch_types=[
            pltpu.VMEM((qpw, NSLOT), jnp.int32),
            pltpu.VMEM((qpw * NSLOT, SEG), jnp.float32),
            pltpu.SemaphoreType.DMA,
        ],
    )
    def k(table_hbm, seg_hbm, out_hbm, segv, rows, sem):
        wid = lax.axis_index("s") * nc + lax.axis_index("c")
        qbase = wid * qpw
        pltpu.sync_copy(seg_hbm.at[pl.ds(qbase, qpw)], segv)
        copies = []
        for i in range(qpw):
            q = qbase + i
            flat = segv[i, :] * 8 + ((q // 8) * row_stride + q % 8)
            copies.append(pltpu.async_copy(
                table_hbm.at[flat],
                rows.at[pl.ds(i * NSLOT, NSLOT), :], sem))
        for c in copies:
            c.wait()
        pltpu.sync_copy(rows, out_hbm.at[pl.ds(qbase * NSLOT, qpw * NSLOT)])

    return k(table, seg_ids).reshape(q_total, NSLOT, SEG)


def _phase_d_kernel(g_ref, seg_ref, vals_ref, idx_ref, *, k_real):
    qb = seg_ref.shape[0]
    seg = seg_ref[...]
    lane = jax.lax.broadcasted_iota(jnp.int32, (qb, SEG), 1)
    g3 = g_ref[...]
    cand = jnp.concatenate([g3[:, t, :] for t in range(NPOP)], axis=1)
    gidx = jnp.concatenate(
        [seg[:, t:t + 1] * SEG + lane for t in range(NPOP)], axis=1)
    cand = jnp.where(gidx < k_real, cand, NEG_INF)
    vals = []
    idxs = []
    for i in range(TOPK):
        mx = jnp.max(cand, axis=1, keepdims=True)
        sel = jnp.min(jnp.where(cand == mx, gidx, BIG_I32), axis=1,
                      keepdims=True)
        cand = jnp.where(gidx == sel, NEG_INF, cand)
        vals.append(mx)
        idxs.append(sel)
    vals_ref[...] = jnp.concatenate(vals, axis=1)
    idx_ref[...] = jnp.concatenate(idxs, axis=1)


def kernel(query, memory_embeddings, top_k):
    del top_k
    q_total, d = query.shape
    k_real = memory_embeddings.shape[0]
    nkc = -(-k_real // KC)
    k_pad = nkc * KC
    nseg = k_pad // SEG
    if k_pad != k_real:
        mem = jnp.pad(memory_embeddings, ((0, k_pad - k_real), (0, 0)))
    else:
        mem = memory_embeddings
    query = query / jnp.maximum(
        jnp.linalg.norm(query, axis=-1, keepdims=True), EPS)
    mem = mem / jnp.maximum(
        jnp.linalg.norm(mem, axis=-1, keepdims=True), EPS)
    qb = min(256, q_total)

    sims3, seg_ids = pl.pallas_call(
        functools.partial(_phase_a_kernel, k_real=k_real, nkc=nkc, qb=qb),
        grid=(nkc, q_total // qb),
        in_specs=[
            pl.BlockSpec((qb, d), lambda j, i: (i, 0)),
            pl.BlockSpec((KC, d), lambda j, i: (j, 0)),
        ],
        out_specs=[
            pl.BlockSpec((qb // 8, SEG_PER_KC * 8, SEG),
                         lambda j, i: (i, j, 0)),
            pl.BlockSpec((qb, NSLOT), lambda j, i: (i, 0)),
        ],
        out_shape=[
            jax.ShapeDtypeStruct((q_total // 8, nseg * 8, SEG), jnp.float32),
            jax.ShapeDtypeStruct((q_total, NSLOT), jnp.int32),
        ],
        scratch_shapes=[
            pltpu.VMEM((q_total, NSLOT), jnp.float32),
            pltpu.VMEM((q_total, NSLOT), jnp.int32),
        ],
    )(query, mem)

    table = sims3.reshape(q_total * nseg, SEG)
    g = _gather_sc(table, seg_ids, nseg)

    vals, idx = pl.pallas_call(
        functools.partial(_phase_d_kernel, k_real=k_real),
        grid=(q_total // qb,),
        in_specs=[
            pl.BlockSpec((qb, NSLOT, SEG), lambda i: (i, 0, 0)),
            pl.BlockSpec((qb, NSLOT), lambda i: (i, 0)),
        ],
        out_specs=[
            pl.BlockSpec((qb, TOPK), lambda i: (i, 0)),
            pl.BlockSpec((qb, TOPK), lambda i: (i, 0)),
        ],
        out_shape=[
            jax.ShapeDtypeStruct((q_total, TOPK), jnp.float32),
            jax.ShapeDtypeStruct((q_total, TOPK), jnp.int32),
        ],
    )(g, seg_ids)
    return vals, idx


# speedup vs baseline: 1.1368x; 1.0004x over previous
import functools

import jax
import jax.numpy as jnp
from jax import lax
from jax.experimental import pallas as pl
from jax.experimental.pallas import tpu as pltpu
from jax.experimental.pallas import tpu_sc as plsc

NEG_INF = float("-inf")
BIG_I32 = 2**31 - 1
EPS = 1e-8
TOPK = 10
NPOP = 11
SEG = 128
KC = 10240
SEG_PER_KC = KC // SEG
NSLOT = 16


def _phase_a_kernel(q_ref, m_ref, sims_ref, seg_ref, rv_ref, ri_ref,
                    *, k_real, nkc, qb):
    j = pl.program_id(0)
    i = pl.program_id(1)
    rv_ref = rv_ref.at[pl.ds(i * qb, qb), :]
    ri_ref = ri_ref.at[pl.ds(i * qb, qb), :]

    @pl.when(j == 0)
    def _init():
        rv_ref[...] = jnp.full((qb, NSLOT), NEG_INF, jnp.float32)
        ri_ref[...] = jnp.zeros((qb, NSLOT), jnp.int32)

    qn = q_ref[...]
    c = j

    nsub = 4
    sub_seg = SEG_PER_KC // nsub
    sub_kc = KC // nsub
    maxes = []

    def _process(sims, t):
        for s0 in range(sub_seg):
            s = t * sub_seg + s0
            piece = sims[:, s0 * SEG:(s0 + 1) * SEG]
            sims_ref[:, s * 8:(s + 1) * 8, :] = piece.reshape(qb // 8, 8,
                                                              SEG)
            maxes.append(jnp.max(piece, axis=1, keepdims=True))

    def _mm(t):
        return jax.lax.dot_general(
            qn, m_ref[pl.ds(t * sub_kc, sub_kc), :],
            (((1,), (1,)), ((), ())),
            preferred_element_type=jnp.float32)

    prev = _mm(0)
    for t in range(1, nsub):
        cur = _mm(t)
        _process(prev, t - 1)
        prev = cur
    _process(prev, nsub - 1)
    segm = jnp.concatenate(maxes, axis=1)

    gseg = c * SEG_PER_KC + jax.lax.broadcasted_iota(
        jnp.int32, (qb, SEG_PER_KC), 1)
    segm = jnp.where(gseg * SEG < k_real, segm, NEG_INF)
    aug_v = jnp.concatenate([rv_ref[...], segm], axis=1)
    aug_i = jnp.concatenate([ri_ref[...], gseg], axis=1)
    lane = jax.lax.broadcasted_iota(jnp.int32, (qb, NSLOT), 1)

    new_v = jnp.full((qb, NSLOT), NEG_INF, jnp.float32)
    new_i = jnp.zeros((qb, NSLOT), jnp.int32)
    for i in range(NPOP):
        mx = jnp.max(aug_v, axis=1, keepdims=True)
        sel = jnp.min(jnp.where(aug_v == mx, aug_i, BIG_I32), axis=1,
                      keepdims=True)
        aug_v = jnp.where(aug_i == sel, NEG_INF, aug_v)
        new_v = jnp.where(lane == i, mx, new_v)
        new_i = jnp.where(lane == i, sel, new_i)

    rv_ref[...] = new_v
    ri_ref[...] = new_i

    @pl.when(j == nkc - 1)
    def _emit():
        seg_ref[...] = new_i


def _gather_sc(table, seg_ids, nseg):
    q_total = seg_ids.shape[0]
    info = plsc.get_sparse_core_info()
    nc, ns = info.num_cores, info.num_subcores
    nw = nc * ns
    qpw = q_total // nw
    row_stride = nseg * 8
    mesh = plsc.VectorSubcoreMesh(core_axis_name="c", subcore_axis_name="s")

    @functools.partial(
        pl.kernel, mesh=mesh,
        out_type=jax.ShapeDtypeStruct((q_total * NSLOT, SEG), jnp.float32),
        scratch_types=[
            pltpu.VMEM((qpw, NSLOT), jnp.int32),
            pltpu.VMEM((qpw * NSLOT, SEG), jnp.float32),
            pltpu.SemaphoreType.DMA,
        ],
    )
    def k(table_hbm, seg_hbm, out_hbm, segv, rows, sem):
        wid = lax.axis_index("s") * nc + lax.axis_index("c")
        qbase = wid * qpw
        pltpu.sync_copy(seg_hbm.at[pl.ds(qbase, qpw)], segv)
        copies = []
        for i in range(qpw):
            q = qbase + i
            flat = segv[i, :] * 8 + ((q // 8) * row_stride + q % 8)
            copies.append(pltpu.async_copy(
                table_hbm.at[flat],
                rows.at[pl.ds(i * NSLOT, NSLOT), :], sem))
        for c in copies:
            c.wait()
        pltpu.sync_copy(rows, out_hbm.at[pl.ds(qbase * NSLOT, qpw * NSLOT)])

    return k(table, seg_ids).reshape(q_total, NSLOT, SEG)


def _phase_d_kernel(g_ref, seg_ref, vals_ref, idx_ref, *, k_real):
    qb = seg_ref.shape[0]
    seg = seg_ref[...]
    lane = jax.lax.broadcasted_iota(jnp.int32, (qb, SEG), 1)
    g3 = g_ref[...]
    cand = jnp.concatenate([g3[:, t, :] for t in range(NPOP)], axis=1)
    gidx = jnp.concatenate(
        [seg[:, t:t + 1] * SEG + lane for t in range(NPOP)], axis=1)
    cand = jnp.where(gidx < k_real, cand, NEG_INF)
    vals = []
    idxs = []
    for i in range(TOPK):
        mx = jnp.max(cand, axis=1, keepdims=True)
        sel = jnp.min(jnp.where(cand == mx, gidx, BIG_I32), axis=1,
                      keepdims=True)
        cand = jnp.where(gidx == sel, NEG_INF, cand)
        vals.append(mx)
        idxs.append(sel)
    vals_ref[...] = jnp.concatenate(vals, axis=1)
    idx_ref[...] = jnp.concatenate(idxs, axis=1)


def kernel(query, memory_embeddings, top_k):
    del top_k
    q_total, d = query.shape
    k_real = memory_embeddings.shape[0]
    nkc = -(-k_real // KC)
    k_pad = nkc * KC
    nseg = k_pad // SEG
    if k_pad != k_real:
        mem = jnp.pad(memory_embeddings, ((0, k_pad - k_real), (0, 0)))
    else:
        mem = memory_embeddings
    query = query / jnp.maximum(
        jnp.linalg.norm(query, axis=-1, keepdims=True), EPS)
    mem = mem / jnp.maximum(
        jnp.linalg.norm(mem, axis=-1, keepdims=True), EPS)
    qb = min(256, q_total)

    sims3, seg_ids = pl.pallas_call(
        functools.partial(_phase_a_kernel, k_real=k_real, nkc=nkc, qb=qb),
        grid=(nkc, q_total // qb),
        in_specs=[
            pl.BlockSpec((qb, d), lambda j, i: (i, 0)),
            pl.BlockSpec((KC, d), lambda j, i: (j, 0)),
        ],
        out_specs=[
            pl.BlockSpec((qb // 8, SEG_PER_KC * 8, SEG),
                         lambda j, i: (i, j, 0)),
            pl.BlockSpec((qb, NSLOT), lambda j, i: (i, 0)),
        ],
        out_shape=[
            jax.ShapeDtypeStruct((q_total // 8, nseg * 8, SEG), jnp.float32),
            jax.ShapeDtypeStruct((q_total, NSLOT), jnp.int32),
        ],
        scratch_shapes=[
            pltpu.VMEM((q_total, NSLOT), jnp.float32),
            pltpu.VMEM((q_total, NSLOT), jnp.int32),
        ],
    )(query, mem)

    table = sims3.reshape(q_total * nseg, SEG)
    g = _gather_sc(table, seg_ids, nseg)

    vals, idx = pl.pallas_call(
        functools.partial(_phase_d_kernel, k_real=k_real),
        grid=(q_total // qb,),
        in_specs=[
            pl.BlockSpec((qb, NSLOT, SEG), lambda i: (i, 0, 0)),
            pl.BlockSpec((qb, NSLOT), lambda i: (i, 0)),
        ],
        out_specs=[
            pl.BlockSpec((qb, TOPK), lambda i: (i, 0)),
            pl.BlockSpec((qb, TOPK), lambda i: (i, 0)),
        ],
        out_shape=[
            jax.ShapeDtypeStruct((q_total, TOPK), jnp.float32),
            jax.ShapeDtypeStruct((q_total, TOPK), jnp.int32),
        ],
    )(g, seg_ids)
    return vals, idx
